# baseline (speedup 1.0000x reference)
import jax
import jax.numpy as jnp
from jax import lax
from jax.experimental import pallas as pl
from jax.experimental.pallas import tpu as pltpu
from jax.experimental.pallas import tpu_sc as plsc

N = 10000
E = 320000
D = 128
H = 64
A = 8

NC = 2
NS = 16
NW = NC * NS

EB = 128
NB_TOT = E // EB
NB_LO = NB_TOT // NW
NB_REM = NB_TOT - NB_LO * NW
RC = 128
N_CHUNKS = [(k * RC, min(RC, N - k * RC)) for k in range((N + RC - 1) // RC)]
DR = (N + D - 1) // D


def _make_sc_scatter(with_deg):
  mesh = plsc.VectorSubcoreMesh(
      core_axis_name="c", subcore_axis_name="s",
      num_cores=NC, num_subcores=NS)

  out_type = [jax.ShapeDtypeStruct((NC, N, D), jnp.float32)]
  scratch = [
      pltpu.VMEM((2, EB), jnp.int32),
      pltpu.VMEM((2, EB), jnp.int32),
      pltpu.VMEM((EB, D), jnp.float32),
      pltpu.VMEM((EB, D), jnp.float32),
      pltpu.SemaphoreType.DMA,
      pltpu.SemaphoreType.DMA,
      pltpu.SemaphoreType.DMA,
      pltpu.SemaphoreType.DMA,
      pltpu.VMEM_SHARED((N, D), jnp.float32),
  ]
  if with_deg:
    out_type.append(jax.ShapeDtypeStruct((NW, DR, D), jnp.float32))
    scratch.append(pltpu.VMEM((DR, D), jnp.float32))

  def body(y, eidx, part, *rest):
    if with_deg:
      (degp, idx_a, idx_b, rows_a, rows_b,
       sem_ia, sem_ib, sem_ga, sem_gb, acc, deg_v) = rest
    else:
      (idx_a, idx_b, rows_a, rows_b,
       sem_ia, sem_ib, sem_ga, sem_gb, acc) = rest
    c = lax.axis_index("c")
    s = lax.axis_index("s")
    w = s * NC + c
    base = w * NB_LO

    def idx_start(g, idx_i, sem):
      pltpu.async_copy(eidx.at[g], idx_i, sem)

    def idx_wait(idx_i, sem):
      pltpu.make_async_copy(eidx.at[0], idx_i, sem).wait()

    def gather_start(idx_i, rows, sem):
      pltpu.async_copy(y.at[idx_i.at[0]], rows, sem)

    def gather_wait(idx_i, rows, sem):
      pltpu.make_async_copy(y.at[idx_i.at[0]], rows, sem).wait()

    idx_start(base, idx_a, sem_ia)
    idx_start(base + 1, idx_b, sem_ib)

    zv = jnp.zeros((16,), jnp.float32)

    def zrow(i, carry):
      for jj in range(D // 16):
        rows_a[i, pl.ds(jj * 16, 16)] = zv
      return carry

    lax.fori_loop(0, EB, zrow, 0)

    for k, (off, sz) in enumerate(N_CHUNKS):
      @pl.when(s == k % NS)
      def _():
        pltpu.async_copy(rows_a.at[pl.ds(0, sz)], acc.at[pl.ds(off, sz)],
                         sem_gb)
    for k, (off, sz) in enumerate(N_CHUNKS):
      @pl.when(s == k % NS)
      def _():
        pltpu.make_async_copy(rows_a.at[pl.ds(0, sz)],
                              acc.at[pl.ds(off, sz)], sem_gb).wait()

    if with_deg:
      def zdrow(i, carry):
        for jj in range(D // 16):
          deg_v[i, pl.ds(jj * 16, 16)] = zv
        return carry

      lax.fori_loop(0, DR, zdrow, 0)
      one16 = jnp.ones((16,), jnp.float32)

    def deg_update(idx_i):
      if with_deg:
        for jj in range(EB // 16):
          idx = idx_i[1, pl.ds(jj * 16, 16)]
          plsc.addupdate_scatter(
              deg_v, [lax.shift_right_logical(idx, 7),
                      lax.bitwise_and(idx, 127)], one16)

    plsc.subcore_barrier()

    idx_wait(idx_a, sem_ia)
    gather_start(idx_a, rows_a, sem_ga)

    def outer(t, carry):
      j0 = 2 * t
      j1 = j0 + 1
      gather_wait(idx_a, rows_a, sem_ga)
      idx_wait(idx_b, sem_ib)
      gather_start(idx_b, rows_b, sem_gb)
      pltpu.sync_copy(rows_a, acc.at[idx_a.at[1]], add=True)
      deg_update(idx_a)
      idx_start(base + j0 + 2, idx_a, sem_ia)
      gather_wait(idx_b, rows_b, sem_gb)
      idx_wait(idx_a, sem_ia)
      gather_start(idx_a, rows_a, sem_ga)
      pltpu.sync_copy(rows_b, acc.at[idx_b.at[1]], add=True)
      deg_update(idx_b)
      idx_start(base + j1 + 2, idx_b, sem_ib)
      return carry

    lax.fori_loop(0, NB_LO // 2, outer, 0)

    idx_wait(idx_b, sem_ib)
    gather_wait(idx_a, rows_a, sem_ga)

    @pl.when(w < NB_REM)
    def _():
      idx_start(NW * NB_LO + w, idx_a, sem_ia)
      idx_wait(idx_a, sem_ia)
      gather_start(idx_a, rows_a, sem_ga)
      gather_wait(idx_a, rows_a, sem_ga)
      pltpu.sync_copy(rows_a, acc.at[idx_a.at[1]], add=True)
      deg_update(idx_a)

    if with_deg:
      pltpu.async_copy(deg_v, degp.at[w], sem_ga)

    plsc.subcore_barrier()

    for k, (off, sz) in enumerate(N_CHUNKS):
      @pl.when(s == k % NS)
      def _():
        pltpu.async_copy(acc.at[pl.ds(off, sz)], part.at[c].at[pl.ds(off, sz)],
                         sem_gb)
    if with_deg:
      pltpu.make_async_copy(deg_v, degp.at[w], sem_ga).wait()
    for k, (off, sz) in enumerate(N_CHUNKS):
      @pl.when(s == k % NS)
      def _():
        pltpu.make_async_copy(acc.at[pl.ds(off, sz)],
                              part.at[c].at[pl.ds(off, sz)], sem_gb).wait()

  return pl.kernel(
      body,
      out_type=tuple(out_type) if with_deg else out_type[0],
      mesh=mesh,
      scratch_types=scratch,
      compiler_params=pltpu.CompilerParams(needs_layout_passes=False),
  )


_sc_scatter_deg = _make_sc_scatter(True)
_sc_scatter = _make_sc_scatter(False)


BM = 1000


def _mm_body(x_ref, w_ref, b_ref, o_ref):
  o_ref[...] = (
      jnp.dot(x_ref[...], w_ref[...], preferred_element_type=jnp.float32)
      + b_ref[...])


_mm1 = pl.pallas_call(
    _mm_body,
    grid=(N // BM,),
    in_specs=[
        pl.BlockSpec((BM, D), lambda i: (i, 0)),
        pl.BlockSpec((D, D), lambda i: (0, 0)),
        pl.BlockSpec((1, D), lambda i: (0, 0)),
    ],
    out_specs=pl.BlockSpec((BM, D), lambda i: (i, 0)),
    out_shape=jax.ShapeDtypeStruct((N, D), jnp.float32),
)


def _degsum_body(p_ref, o_ref):
  o_ref[...] = jnp.sum(p_ref[...], axis=0)


_degsum = pl.pallas_call(
    _degsum_body,
    in_specs=[pl.BlockSpec((NW, DR, D), lambda: (0, 0, 0))],
    out_specs=pl.BlockSpec((DR, D), lambda: (0, 0)),
    out_shape=jax.ShapeDtypeStruct((DR, D), jnp.float32),
)


def _mid_body(p_ref, g_ref, w_ref, b_ref, o_ref):
  ssum = p_ref[0] + p_ref[1]
  deg = jnp.maximum(g_ref[...], 1.0)
  h = jnp.maximum(ssum / deg, 0.0)
  o_ref[...] = (
      jnp.dot(h, w_ref[...], preferred_element_type=jnp.float32)
      + b_ref[...])


_mid = pl.pallas_call(
    _mid_body,
    grid=(N // BM,),
    in_specs=[
        pl.BlockSpec((2, BM, D), lambda i: (0, i, 0)),
        pl.BlockSpec((BM, 1), lambda i: (i, 0)),
        pl.BlockSpec((D, D), lambda i: (0, 0)),
        pl.BlockSpec((1, D), lambda i: (0, 0)),
    ],
    out_specs=pl.BlockSpec((BM, D), lambda i: (i, 0)),
    out_shape=jax.ShapeDtypeStruct((N, D), jnp.float32),
)


def _head_body(p_ref, g_ref, w1_ref, b1_ref, w2_ref, b2_ref, o_ref):
  ssum = p_ref[0] + p_ref[1]
  deg = jnp.maximum(g_ref[...], 1.0)
  h2 = jnp.maximum(ssum / deg, 0.0)
  hid = jnp.maximum(
      jnp.dot(h2, w1_ref[...], preferred_element_type=jnp.float32)
      + b1_ref[...], 0.0)
  o_ref[...] = jnp.tanh(
      jnp.dot(hid, w2_ref[...], preferred_element_type=jnp.float32)
      + b2_ref[...])


_head = pl.pallas_call(
    _head_body,
    grid=(N // BM,),
    in_specs=[
        pl.BlockSpec((2, BM, D), lambda i: (0, i, 0)),
        pl.BlockSpec((BM, 1), lambda i: (i, 0)),
        pl.BlockSpec((D, H), lambda i: (0, 0)),
        pl.BlockSpec((1, H), lambda i: (0, 0)),
        pl.BlockSpec((H, A), lambda i: (0, 0)),
        pl.BlockSpec((1, A), lambda i: (0, 0)),
    ],
    out_specs=pl.BlockSpec((BM, A), lambda i: (i, 0)),
    out_shape=jax.ShapeDtypeStruct((N, A), jnp.float32),
)


def kernel(x, edge_index, W1, b1, W2, b2, Wm1, bm1, Wm2, bm2):
  eidx = edge_index.reshape(2, NB_TOT, EB).transpose(1, 0, 2)
  y1 = _mm1(x, W1, b1.reshape(1, D))
  part1, degp = _sc_scatter_deg(y1, eidx)
  deg_col = _degsum(degp).reshape(DR * D)[:N].reshape(N, 1)
  y2 = _mid(part1, deg_col, W2, b2.reshape(1, D))
  part2 = _sc_scatter(y2, eidx)
  return _head(part2, deg_col, Wm1, bm1.reshape(1, H), Wm2, bm2.reshape(1, A))
